# baseline (speedup 1.0000x reference)
import functools

import jax
import jax.numpy as jnp
from jax import lax
from jax.experimental import pallas as pl
from jax.experimental.pallas import tpu as pltpu
from jax.experimental.pallas import tpu_sc as plsc

_NC = 2
_NS = 16
_NW = _NC * _NS
_CHUNK = 64


def _logits_body(x_ref, w_ref, w1_ref, w2_ref, o_ref, w1b_ref, w2b_ref):
    x = x_ref[...]
    o_ref[...] = jnp.dot(x.astype(jnp.bfloat16),
                         w_ref[...].astype(jnp.bfloat16)[:, None],
                         preferred_element_type=jnp.float32)
    w1b_ref[...] = w1_ref[...].astype(jnp.bfloat16)
    w2b_ref[...] = w2_ref[...].astype(jnp.bfloat16)


def _router_logits(h2, w, w1, w2):
    n, d = h2.shape
    dff = w1.shape[1]
    rb = 1024
    nblk = n // rb
    fb = dff // nblk
    out, w1b, w2b = pl.pallas_call(
        _logits_body,
        grid=(nblk,),
        in_specs=[
            pl.BlockSpec((rb, d), lambda i: (i, 0)),
            pl.BlockSpec((d,), lambda i: (0,)),
            pl.BlockSpec((d, fb), lambda i: (0, i)),
            pl.BlockSpec((fb, d), lambda i: (i, 0)),
        ],
        out_specs=[
            pl.BlockSpec((rb, 1), lambda i: (i, 0)),
            pl.BlockSpec((d, fb), lambda i: (0, i)),
            pl.BlockSpec((fb, d), lambda i: (i, 0)),
        ],
        out_shape=[
            jax.ShapeDtypeStruct((n, 1), jnp.float32),
            jax.ShapeDtypeStruct((d, dff), jnp.bfloat16),
            jax.ShapeDtypeStruct((dff, d), jnp.bfloat16),
        ],
    )(h2, w, w1, w2)
    return out.reshape(n), w1b, w2b


def _ffn_body(x_ref, w1_ref, b1_ref, w2_ref, b2_ref, v_ref, o_ref):
    x = x_ref[...]
    h = jnp.dot(x.astype(jnp.bfloat16), w1_ref[...],
                preferred_element_type=jnp.float32)
    h = jax.nn.gelu((h + b1_ref[...][None, :]).astype(jnp.bfloat16))
    y = jnp.dot(h, w2_ref[...], preferred_element_type=jnp.float32)
    y = y + b2_ref[...][None, :]
    o_ref[...] = x + v_ref[...] * y


def _ffn(x, w1b, b1, w2b, b2, vals):
    n, d = x.shape
    dff = w1b.shape[1]
    bm = 512
    return pl.pallas_call(
        _ffn_body,
        grid=(n // bm,),
        in_specs=[
            pl.BlockSpec((bm, d), lambda i: (i, 0)),
            pl.BlockSpec((d, dff), lambda i: (0, 0)),
            pl.BlockSpec((dff,), lambda i: (0,)),
            pl.BlockSpec((dff, d), lambda i: (0, 0)),
            pl.BlockSpec((d,), lambda i: (0,)),
            pl.BlockSpec((bm, 1), lambda i: (i, 0)),
        ],
        out_specs=pl.BlockSpec((bm, d), lambda i: (i, 0)),
        out_shape=jax.ShapeDtypeStruct((n, d), jnp.float32),
    )(x, w1b, b1, w2b, b2, vals)


def _sc_mesh():
    return plsc.VectorSubcoreMesh(core_axis_name="c", subcore_axis_name="s",
                                  num_cores=_NC, num_subcores=_NS)


def _make_copy(n_rows, d):
    rows_per_w = n_rows // _NW

    @functools.partial(
        pl.kernel,
        out_type=jax.ShapeDtypeStruct((n_rows, d), jnp.float32),
        mesh=_sc_mesh(),
    )
    def copy_k(src_hbm, out_hbm):
        wid = lax.axis_index("s") * _NC + lax.axis_index("c")
        base = wid * rows_per_w
        pltpu.sync_copy(src_hbm.at[pl.ds(base, rows_per_w)],
                        out_hbm.at[pl.ds(base, rows_per_w)])

    return copy_k


def _make_gather(n_rows, d):
    rows_per_w = n_rows // _NW
    n_chunks = rows_per_w // _CHUNK

    @functools.partial(
        pl.kernel,
        out_type=jax.ShapeDtypeStruct((n_rows, d), jnp.float32),
        mesh=_sc_mesh(),
        scratch_types=[
            pltpu.VMEM((_CHUNK,), jnp.int32),
            pltpu.VMEM((_CHUNK, d), jnp.float32),
            pltpu.SemaphoreType.DMA,
        ],
    )
    def gather_k(src_hbm, idx_hbm, out_hbm, idx_v, rows_v, sem):
        wid = lax.axis_index("s") * _NC + lax.axis_index("c")
        base = wid * rows_per_w
        for c in range(n_chunks):
            off = base + c * _CHUNK
            pltpu.sync_copy(idx_hbm.at[pl.ds(off, _CHUNK)], idx_v)
            pltpu.async_copy(src_hbm.at[idx_v], rows_v, sem).wait()
            pltpu.sync_copy(rows_v, out_hbm.at[pl.ds(off, _CHUNK)])

    return gather_k


def _make_scatter(n_rows, d):
    rows_per_w = n_rows // _NW
    n_chunks = rows_per_w // _CHUNK

    @functools.partial(
        pl.kernel,
        out_type=(),
        mesh=_sc_mesh(),
        scratch_types=[
            pltpu.VMEM((_CHUNK,), jnp.int32),
            pltpu.VMEM((_CHUNK, d), jnp.float32),
            pltpu.SemaphoreType.DMA,
        ],
    )
    def scatter_k(out_hbm, rows_hbm, idx_hbm, idx_v, rows_v, sem):
        wid = lax.axis_index("s") * _NC + lax.axis_index("c")
        base = wid * rows_per_w
        for c in range(n_chunks):
            off = base + c * _CHUNK
            pltpu.sync_copy(idx_hbm.at[pl.ds(off, _CHUNK)], idx_v)
            pltpu.sync_copy(rows_hbm.at[pl.ds(off, _CHUNK)], rows_v)
            pltpu.async_copy(rows_v, out_hbm.at[idx_v], sem).wait()

    return scatter_k


def kernel(hidden, router_weight, router_bias, W1, b1, W2, b2):
    b, s, d = hidden.shape
    k = s // 2
    n = b * k
    h2 = hidden.reshape(b * s, d)

    h2c = _make_copy(b * s, d)(h2)
    logits, w1b, w2b = _router_logits(h2, router_weight, W1, W2)
    probs = jax.nn.sigmoid(logits.reshape(b, s) + router_bias)
    top_vals, top_idx = lax.top_k(probs, k)
    idx_flat = (top_idx.astype(jnp.int32)
                + (jnp.arange(b, dtype=jnp.int32) * s)[:, None]).reshape(n)

    out_ref = jax.new_ref(h2c)

    vals = top_vals.reshape(n, 1)

    gather_k = _make_gather(k, d)
    scatter_k = _make_scatter(k, d)
    for i in range(b):
        sl = pl.ds(i * k, k)
        g = gather_k(h2, idx_flat[sl])
        rows = _ffn(g, w1b, b1, w2b, b2, vals[sl, :])
        scatter_k(out_ref, rows, idx_flat[sl])

    return out_ref[...].reshape(b, s, d)


# speedup vs baseline: 7.6002x; 7.6002x over previous
import functools

import jax
import jax.numpy as jnp
from jax import lax
from jax.experimental import pallas as pl
from jax.experimental.pallas import tpu as pltpu
from jax.experimental.pallas import tpu_sc as plsc

_NC = 2
_NS = 16
_NW = _NC * _NS
_CHUNK = 64


def _logits_body(x_ref, w_ref, w1_ref, w2_ref, o_ref, w1b_ref, w2b_ref):
    x = x_ref[...]
    o_ref[...] = jnp.dot(x.astype(jnp.bfloat16),
                         w_ref[...].astype(jnp.bfloat16)[:, None],
                         preferred_element_type=jnp.float32)
    w1b_ref[...] = w1_ref[...].astype(jnp.bfloat16)
    w2b_ref[...] = w2_ref[...].astype(jnp.bfloat16)


def _router_logits(h2, w, w1, w2):
    n, d = h2.shape
    dff = w1.shape[1]
    rb = 1024
    nblk = n // rb
    fb = dff // nblk
    out, w1b, w2b = pl.pallas_call(
        _logits_body,
        grid=(nblk,),
        in_specs=[
            pl.BlockSpec((rb, d), lambda i: (i, 0)),
            pl.BlockSpec((d,), lambda i: (0,)),
            pl.BlockSpec((d, fb), lambda i: (0, i)),
            pl.BlockSpec((fb, d), lambda i: (i, 0)),
        ],
        out_specs=[
            pl.BlockSpec((rb, 1), lambda i: (i, 0)),
            pl.BlockSpec((d, fb), lambda i: (0, i)),
            pl.BlockSpec((fb, d), lambda i: (i, 0)),
        ],
        out_shape=[
            jax.ShapeDtypeStruct((n, 1), jnp.float32),
            jax.ShapeDtypeStruct((d, dff), jnp.bfloat16),
            jax.ShapeDtypeStruct((dff, d), jnp.bfloat16),
        ],
    )(h2, w, w1, w2)
    return out.reshape(n), w1b, w2b


def _ffn_body(x_ref, w1_ref, b1_ref, w2_ref, b2_ref, v_ref, o_ref):
    x = x_ref[...]
    h = jnp.dot(x.astype(jnp.bfloat16), w1_ref[...],
                preferred_element_type=jnp.float32)
    h = jax.nn.gelu((h + b1_ref[...][None, :]).astype(jnp.bfloat16))
    y = jnp.dot(h, w2_ref[...], preferred_element_type=jnp.float32)
    y = y + b2_ref[...][None, :]
    o_ref[...] = x + v_ref[...] * y


def _ffn(x, w1b, b1, w2b, b2, vals):
    n, d = x.shape
    dff = w1b.shape[1]
    bm = 512
    return pl.pallas_call(
        _ffn_body,
        grid=(n // bm,),
        in_specs=[
            pl.BlockSpec((bm, d), lambda i: (i, 0)),
            pl.BlockSpec((d, dff), lambda i: (0, 0)),
            pl.BlockSpec((dff,), lambda i: (0,)),
            pl.BlockSpec((dff, d), lambda i: (0, 0)),
            pl.BlockSpec((d,), lambda i: (0,)),
            pl.BlockSpec((bm, 1), lambda i: (i, 0)),
        ],
        out_specs=pl.BlockSpec((bm, d), lambda i: (i, 0)),
        out_shape=jax.ShapeDtypeStruct((n, d), jnp.float32),
    )(x, w1b, b1, w2b, b2, vals)


def _sc_mesh():
    return plsc.VectorSubcoreMesh(core_axis_name="c", subcore_axis_name="s",
                                  num_cores=_NC, num_subcores=_NS)


def _make_copy(n_rows, d):
    rows_per_w = n_rows // _NW

    cchunk = 128
    n_chunks = rows_per_w // cchunk

    @functools.partial(
        pl.kernel,
        out_type=jax.ShapeDtypeStruct((n_rows, d), jnp.float32),
        mesh=_sc_mesh(),
        scratch_types=[pltpu.VMEM((cchunk, d), jnp.float32)],
    )
    def copy_k(src_hbm, out_hbm, rows_v):
        wid = lax.axis_index("s") * _NC + lax.axis_index("c")
        base = wid * rows_per_w
        for c in range(n_chunks):
            off = base + c * cchunk
            pltpu.sync_copy(src_hbm.at[pl.ds(off, cchunk)], rows_v)
            pltpu.sync_copy(rows_v, out_hbm.at[pl.ds(off, cchunk)])

    return copy_k


def _make_gather(n_rows, d):
    rows_per_w = n_rows // _NW
    n_chunks = rows_per_w // _CHUNK

    @functools.partial(
        pl.kernel,
        out_type=jax.ShapeDtypeStruct((n_rows, d), jnp.float32),
        mesh=_sc_mesh(),
        scratch_types=[
            pltpu.VMEM((_CHUNK,), jnp.int32),
            pltpu.VMEM((_CHUNK, d), jnp.float32),
            pltpu.SemaphoreType.DMA,
        ],
    )
    def gather_k(src_hbm, idx_hbm, out_hbm, idx_v, rows_v, sem):
        wid = lax.axis_index("s") * _NC + lax.axis_index("c")
        base = wid * rows_per_w
        for c in range(n_chunks):
            off = base + c * _CHUNK
            pltpu.sync_copy(idx_hbm.at[pl.ds(off, _CHUNK)], idx_v)
            pltpu.async_copy(src_hbm.at[idx_v], rows_v, sem).wait()
            pltpu.sync_copy(rows_v, out_hbm.at[pl.ds(off, _CHUNK)])

    return gather_k


def _make_scatter(n_rows, d):
    rows_per_w = n_rows // _NW
    n_chunks = rows_per_w // _CHUNK

    @functools.partial(
        pl.kernel,
        out_type=(),
        mesh=_sc_mesh(),
        scratch_types=[
            pltpu.VMEM((_CHUNK,), jnp.int32),
            pltpu.VMEM((_CHUNK, d), jnp.float32),
            pltpu.SemaphoreType.DMA,
        ],
    )
    def scatter_k(out_hbm, rows_hbm, idx_hbm, idx_v, rows_v, sem):
        wid = lax.axis_index("s") * _NC + lax.axis_index("c")
        base = wid * rows_per_w
        for c in range(n_chunks):
            off = base + c * _CHUNK
            pltpu.sync_copy(idx_hbm.at[pl.ds(off, _CHUNK)], idx_v)
            pltpu.sync_copy(rows_hbm.at[pl.ds(off, _CHUNK)], rows_v)
            pltpu.async_copy(rows_v, out_hbm.at[idx_v], sem).wait()

    return scatter_k


def kernel(hidden, router_weight, router_bias, W1, b1, W2, b2):
    b, s, d = hidden.shape
    k = s // 2
    n = b * k
    h2 = hidden.reshape(b * s, d)

    h2c = _make_copy(b * s, d)(h2)
    logits, w1b, w2b = _router_logits(h2, router_weight, W1, W2)
    probs = jax.nn.sigmoid(logits.reshape(b, s) + router_bias)
    top_vals, top_idx = lax.top_k(probs, k)
    idx_flat = (top_idx.astype(jnp.int32)
                + (jnp.arange(b, dtype=jnp.int32) * s)[:, None]).reshape(n)

    out_ref = jax.new_ref(h2c)

    vals = top_vals.reshape(n, 1)

    gather_k = _make_gather(k, d)
    scatter_k = _make_scatter(k, d)
    for i in range(b):
        sl = pl.ds(i * k, k)
        g = gather_k(h2, idx_flat[sl])
        rows = _ffn(g, w1b, b1, w2b, b2, vals[sl, :])
        scatter_k(out_ref, rows, idx_flat[sl])

    return out_ref[...].reshape(b, s, d)
